# baseline (speedup 1.0000x reference)
import functools

import jax
import jax.numpy as jnp
from jax import lax
from jax.experimental import pallas as pl
from jax.experimental.pallas import tpu as pltpu
from jax.experimental.pallas import tpu_sc as plsc

N_NODES = 10000
N_PAD = 10240
E_EDGES = 160000
E_PAD = 163840
DUMP_ROW = N_NODES
IN_DIM = 256
HID = 256
HALF = 128
CODE_PAD = 32


def _zero_vmem(ref, rows, width):
    z16 = jnp.zeros((16,), jnp.float32)

    def row(i, _):
        def col(k, _):
            ref[i, pl.ds(k * 16, 16)] = z16
            return 0
        return lax.fori_loop(0, width // 16, col, 0)

    lax.fori_loop(0, rows, row, 0)


def _zero_vmem_1d(ref, n):
    z16 = jnp.zeros((16,), jnp.float32)

    def body(i, _):
        ref[pl.ds(i * 16, 16)] = z16
        return 0

    lax.fori_loop(0, n // 16, body, 0)


def _zero_vmem_bf16(ref, rows, width):
    z32 = jnp.zeros((32,), jnp.bfloat16)

    def row(i, _):
        def col(k, _):
            ref[i, pl.ds(k * 32, 32)] = z32
            return 0
        return lax.fori_loop(0, width // 32, col, 0)

    lax.fori_loop(0, rows, row, 0)


def _make_wide_aggr(with_deg):
    mesh = plsc.VectorSubcoreMesh(core_axis_name="c", subcore_axis_name="s",
                                  num_cores=2, num_subcores=16)

    out_type = [jax.ShapeDtypeStruct((2, N_PAD, HALF), jnp.bfloat16)]
    if with_deg:
        out_type.append(jax.ShapeDtypeStruct((2, 16, N_PAD), jnp.float32))

    scratch = [
        pltpu.VMEM((80, 128), jnp.int32),
        pltpu.VMEM((80, 128), jnp.int32),
        [pltpu.VMEM((128, HALF), jnp.bfloat16) for _ in range(4)],
        pltpu.VMEM((128, HALF), jnp.bfloat16),
        pltpu.VMEM_SHARED((N_PAD, HALF), jnp.bfloat16),
        pltpu.SemaphoreType.DMA,
        pltpu.SemaphoreType.DMA,
    ]
    if with_deg:
        scratch.append(pltpu.VMEM((N_PAD,), jnp.float32))

    def body(h0_hbm, h1_hbm, sidx_hbm, didx_hbm, aggr_out, *rest):
        if with_deg:
            deg_out = rest[0]
            sidx_v, didx_v, rows, zbuf_v, table_sh, gsem, ssem, deg_v = rest[1:]
        else:
            sidx_v, didx_v, rows, zbuf_v, table_sh, gsem, ssem = rest
        c = lax.axis_index("c")
        s = lax.axis_index("s")

        _zero_vmem_bf16(zbuf_v, 128, HALF)
        if with_deg:
            _zero_vmem_1d(deg_v, N_PAD)

        pltpu.sync_copy(sidx_hbm.at[s], sidx_v)
        pltpu.sync_copy(didx_hbm.at[s], didx_v)

        ones16 = jnp.ones((16,), jnp.float32)

        for k in range(5):
            pltpu.sync_copy(zbuf_v, table_sh.at[pl.ds((s * 5 + k) * 128, 128)])
        plsc.subcore_barrier()

        def grp(g, _):
            for hbm, cc in ((h0_hbm, 0), (h1_hbm, 1)):
                @pl.when(c == cc)
                def _(hbm=hbm):
                    descs = [
                        pltpu.async_copy(
                            hbm.at[sidx_v.at[g * 4 + b]], rows[b], gsem)
                        for b in range(4)]
                    for d in descs:
                        d.wait()
            sdescs = [
                pltpu.async_copy(
                    rows[b], table_sh.at[didx_v.at[g * 4 + b]], ssem,
                    add=True)
                for b in range(4)]
            for d in sdescs:
                d.wait()
            if with_deg:
                def dcount(k, _):
                    idx16 = didx_v[g * 4 + k // 8, pl.ds((k % 8) * 16, 16)]
                    plsc.addupdate_scatter(deg_v, [idx16], ones16)
                    return 0
                lax.fori_loop(0, 32, dcount, 0)
            return 0

        lax.fori_loop(0, 20, grp, 0)

        plsc.subcore_barrier()
        pltpu.sync_copy(table_sh.at[pl.ds(s * 640, 640)],
                        aggr_out.at[c].at[pl.ds(s * 640, 640)])
        if with_deg:
            pltpu.sync_copy(deg_v, deg_out.at[c].at[s])

    return pl.kernel(
        body, out_type=out_type, mesh=mesh, scratch_types=scratch,
        compiler_params=pltpu.CompilerParams(needs_layout_passes=False,
                                             use_tc_tiling_on_sc=False))


def _make_narrow_aggr():
    mesh = plsc.VectorSubcoreMesh(core_axis_name="c", subcore_axis_name="s",
                                  num_cores=2, num_subcores=16)

    scratch = [
        pltpu.VMEM((40, 128), jnp.int32),
        pltpu.VMEM((40, 128), jnp.int32),
        [pltpu.VMEM((128, CODE_PAD), jnp.float32) for _ in range(4)],
        pltpu.VMEM((320, CODE_PAD), jnp.float32),
        pltpu.VMEM_SHARED((N_PAD, CODE_PAD), jnp.float32),
        pltpu.SemaphoreType.DMA,
        pltpu.SemaphoreType.DMA,
    ]

    def body(tbl_hbm, sidx_hbm, didx_hbm, aggr_out,
             sidx_v, didx_v, rows, zbuf_v, table_sh, gsem, ssem):
        c = lax.axis_index("c")
        s = lax.axis_index("s")
        w = c * 16 + s

        _zero_vmem(zbuf_v, 320, CODE_PAD)
        pltpu.sync_copy(zbuf_v, table_sh.at[pl.ds(s * 320, 320)])
        pltpu.sync_copy(zbuf_v, table_sh.at[pl.ds(5120 + s * 320, 320)])

        pltpu.sync_copy(sidx_hbm.at[w], sidx_v)
        pltpu.sync_copy(didx_hbm.at[w], didx_v)
        plsc.subcore_barrier()

        def grp(g, _):
            descs = [
                pltpu.async_copy(tbl_hbm.at[sidx_v.at[g * 4 + b]], rows[b],
                                 gsem)
                for b in range(4)]
            for d in descs:
                d.wait()
            sdescs = [
                pltpu.async_copy(rows[b], table_sh.at[didx_v.at[g * 4 + b]],
                                 ssem, add=True)
                for b in range(4)]
            for d in sdescs:
                d.wait()
            return 0

        lax.fori_loop(0, 10, grp, 0)

        plsc.subcore_barrier()
        pltpu.sync_copy(table_sh.at[pl.ds(s * 640, 640)],
                        aggr_out.at[c].at[pl.ds(s * 640, 640)])

    return pl.kernel(
        body,
        out_type=[jax.ShapeDtypeStruct((2, N_PAD, CODE_PAD), jnp.float32)],
        mesh=mesh, scratch_types=scratch,
        compiler_params=pltpu.CompilerParams(needs_layout_passes=False,
                                             use_tc_tiling_on_sc=False))


def _t1_body(x_ref, a_ref, degs_ref, wl1_ref, wr1_ref, b1_ref,
             wl2_ref, wr2_ref, b2_ref, p2_ref, r2_ref, invd_ref):
    deg = jnp.sum(degs_ref[...], axis=0)
    invd = 1.0 / jnp.clip(deg, 1.0, None)
    invd2 = invd[:, None]
    aggr = sum(jnp.dot(a_ref[i].astype(jnp.float32), wl1_ref[i],
                       preferred_element_type=jnp.float32)
               for i in range(2))
    h1 = jax.nn.relu(aggr * invd2
                     + jnp.dot(x_ref[...], wr1_ref[...],
                               preferred_element_type=jnp.float32)
                     + b1_ref[...])
    p2_ref[...] = jnp.dot(h1, wl2_ref[...], preferred_element_type=jnp.float32)
    r2_ref[...] = (jnp.dot(h1, wr2_ref[...], preferred_element_type=jnp.float32)
                   + b2_ref[...])
    invd_ref[...] = invd2


def _t2_body(a2_ref, r2_ref, invd_ref, g_ref, z_ref):
    t = (a2_ref[0] + a2_ref[1]) * invd_ref[...] + r2_ref[...] + g_ref[...]
    parts = []
    for grp in range(2):
        sl = t[:, grp * 10:(grp + 1) * 10]
        m = jnp.max(sl, axis=1, keepdims=True)
        e = jnp.exp(sl - m)
        parts.append(e / jnp.sum(e, axis=1, keepdims=True))
    parts.append(jnp.zeros((N_PAD, CODE_PAD - 20), jnp.float32))
    z_ref[...] = jnp.concatenate(parts, axis=1)


def _t3_body(a3_ref, z_ref, invd_ref, wl3_ref, wr3_ref, b3_ref,
             h3f_ref, h3b_ref):
    aggr = jnp.dot((a3_ref[0] + a3_ref[1]) * invd_ref[...], wl3_ref[...],
                   preferred_element_type=jnp.float32)
    h3 = jax.nn.relu(aggr
                     + jnp.dot(z_ref[...], wr3_ref[...],
                               preferred_element_type=jnp.float32)
                     + b3_ref[...])
    h3f_ref[...] = h3
    for i in range(2):
        h3b_ref[i] = h3[:, i * HALF:(i + 1) * HALF].astype(jnp.bfloat16)


def _t4_body(a4_ref, h3f_ref, invd_ref, wl4_ref, wr4_ref, b4_ref, out_ref):
    aggr = sum(jnp.dot(a4_ref[i].astype(jnp.float32), wl4_ref[i],
                       preferred_element_type=jnp.float32)
               for i in range(2))
    rec = jnp.dot(h3f_ref[...], wr4_ref[...],
                  preferred_element_type=jnp.float32)
    out_ref[...] = aggr * invd_ref[...] + rec + b4_ref[...]


def _tc_call(body, out_shapes):
    return pl.pallas_call(body, out_shape=out_shapes)


ROWB = 2560


def _full(shape):
    nd = len(shape)
    return pl.BlockSpec(shape, lambda i: (0,) * nd)


def _rows(shape):
    nd = len(shape)
    if nd == 2:
        return pl.BlockSpec((ROWB, shape[1]), lambda i: (i, 0))
    return pl.BlockSpec((shape[0], ROWB, shape[2]), lambda i: (0, i, 0))


def kernel(x, edge_index, W_l1, W_r1, b1, W_l2, W_r2, b2,
           W_l3, W_r3, b3, W_l4, W_r4, b4):
    f32 = jnp.float32
    src = edge_index[0].astype(jnp.int32)
    dst = edge_index[1].astype(jnp.int32)
    src_p = jnp.concatenate(
        [src, jnp.zeros((E_PAD - E_EDGES,), jnp.int32)])
    dst_p = jnp.concatenate(
        [dst, jnp.full((E_PAD - E_EDGES,), DUMP_ROW, jnp.int32)])
    sidx16 = src_p.reshape(16, 80, 128)
    didx16 = dst_p.reshape(16, 80, 128)
    sidx32 = src_p.reshape(32, 40, 128)
    didx32 = dst_p.reshape(32, 40, 128)

    x_p = jnp.pad(x.astype(f32), ((0, N_PAD - N_NODES), (0, 0)))
    x_b = x_p.astype(jnp.bfloat16)
    xh = [x_b[:, :HALF], x_b[:, HALF:]]

    wl1s = W_l1.reshape(2, HALF, HID)
    wl2p = jnp.pad(W_l2, ((0, 0), (0, CODE_PAD - 20)))
    wr2p = jnp.pad(W_r2, ((0, 0), (0, CODE_PAD - 20)))
    b2p = jnp.pad(b2, (0, CODE_PAD - 20))[None, :]
    wl3p = jnp.pad(W_l3, ((0, CODE_PAD - 20), (0, 0)))
    wr3p = jnp.pad(W_r3, ((0, CODE_PAD - 20), (0, 0)))
    wl4s = W_l4.reshape(2, HALF, IN_DIM)

    u = jax.random.uniform(jax.random.key(42), (N_NODES, 2, 10), dtype=f32)
    g = -jnp.log(-jnp.log(u + 1e-20)).reshape(N_NODES, 20)
    g_p = jnp.pad(g, ((0, N_PAD - N_NODES), (0, CODE_PAD - 20)))

    wide_deg = _make_wide_aggr(with_deg=True)
    narrow = _make_narrow_aggr()
    wide = _make_wide_aggr(with_deg=False)

    aggr1, degs = wide_deg(xh[0], xh[1], sidx16, didx16)

    p2, r2, invd = pl.pallas_call(
        _t1_body,
        grid=(N_PAD // ROWB,),
        in_specs=[_rows((N_PAD, HID)), _rows((2, N_PAD, HALF)),
                  pl.BlockSpec((16, ROWB), lambda i: (0, i)),
                  _full((2, HALF, HID)), _full((HID, HID)), _full((1, HID)),
                  _full((HID, CODE_PAD)), _full((HID, CODE_PAD)),
                  _full((1, CODE_PAD))],
        out_specs=[_rows((N_PAD, CODE_PAD)), _rows((N_PAD, CODE_PAD)),
                   _rows((N_PAD, 1))],
        out_shape=[jax.ShapeDtypeStruct((N_PAD, CODE_PAD), f32),
                   jax.ShapeDtypeStruct((N_PAD, CODE_PAD), f32),
                   jax.ShapeDtypeStruct((N_PAD, 1), f32)],
    )(x_p, aggr1, degs[0], wl1s, W_r1, b1[None, :], wl2p, wr2p, b2p)

    (a2,) = narrow(p2, sidx32, didx32)

    (z,) = _tc_call(
        _t2_body, [jax.ShapeDtypeStruct((N_PAD, CODE_PAD), f32)]
    )(a2, r2, invd, g_p)

    (a3,) = narrow(z, sidx32, didx32)

    h3f, h3b = _tc_call(
        _t3_body, [jax.ShapeDtypeStruct((N_PAD, HID), f32),
                   jax.ShapeDtypeStruct((2, N_PAD, HALF), jnp.bfloat16)]
    )(a3, z, invd, wl3p, wr3p, b3[None, :])

    (aggr4,) = wide(h3b[0], h3b[1], sidx16, didx16)

    (out,) = pl.pallas_call(
        _t4_body,
        grid=(N_PAD // ROWB,),
        in_specs=[_rows((2, N_PAD, HALF)), _rows((N_PAD, HID)),
                  _rows((N_PAD, 1)), _full((2, HALF, IN_DIM)),
                  _full((HID, IN_DIM)), _full((1, IN_DIM))],
        out_specs=[_rows((N_PAD, IN_DIM))],
        out_shape=[jax.ShapeDtypeStruct((N_PAD, IN_DIM), f32)],
    )(aggr4, h3f, invd, wl4s, W_r4, b4[None, :])

    return out[:N_NODES]


# speedup vs baseline: 1.1547x; 1.1547x over previous
import functools

import jax
import jax.numpy as jnp
from jax import lax
from jax.experimental import pallas as pl
from jax.experimental.pallas import tpu as pltpu
from jax.experimental.pallas import tpu_sc as plsc

N_NODES = 10000
N_PAD = 10240
E_EDGES = 160000
E_PAD = 163840
DUMP_ROW = N_NODES
IN_DIM = 256
HID = 256
HALF = 128
CODE_PAD = 32


def _zero_vmem(ref, rows, width):
    z16 = jnp.zeros((16,), jnp.float32)

    def row(i, _):
        def col(k, _):
            ref[i, pl.ds(k * 16, 16)] = z16
            return 0
        return lax.fori_loop(0, width // 16, col, 0)

    lax.fori_loop(0, rows, row, 0)


def _zero_vmem_1d(ref, n):
    z16 = jnp.zeros((16,), jnp.float32)

    def body(i, _):
        ref[pl.ds(i * 16, 16)] = z16
        return 0

    lax.fori_loop(0, n // 16, body, 0)


def _zero_vmem_bf16(ref, rows, width):
    z32 = jnp.zeros((32,), jnp.bfloat16)

    def row(i, _):
        def col(k, _):
            ref[i, pl.ds(k * 32, 32)] = z32
            return 0
        return lax.fori_loop(0, width // 32, col, 0)

    lax.fori_loop(0, rows, row, 0)


def _make_wide_aggr(with_deg):
    mesh = plsc.VectorSubcoreMesh(core_axis_name="c", subcore_axis_name="s",
                                  num_cores=2, num_subcores=16)

    out_type = [jax.ShapeDtypeStruct((2, N_PAD, HALF), jnp.bfloat16)]
    if with_deg:
        out_type.append(jax.ShapeDtypeStruct((2, 16, N_PAD), jnp.float32))

    scratch = [
        pltpu.VMEM((80, 128), jnp.int32),
        pltpu.VMEM((80, 128), jnp.int32),
        [pltpu.VMEM((128, HALF), jnp.bfloat16) for _ in range(4)],
        pltpu.VMEM((128, HALF), jnp.bfloat16),
        pltpu.VMEM_SHARED((N_PAD, HALF), jnp.bfloat16),
        pltpu.SemaphoreType.DMA,
        pltpu.SemaphoreType.DMA,
    ]
    if with_deg:
        scratch.append(pltpu.VMEM((N_PAD,), jnp.float32))

    def body(h0_hbm, h1_hbm, sidx_hbm, didx_hbm, aggr_out, *rest):
        if with_deg:
            deg_out = rest[0]
            sidx_v, didx_v, rows, zbuf_v, table_sh, gsem, ssem, deg_v = rest[1:]
        else:
            sidx_v, didx_v, rows, zbuf_v, table_sh, gsem, ssem = rest
        c = lax.axis_index("c")
        s = lax.axis_index("s")

        _zero_vmem_bf16(zbuf_v, 128, HALF)
        if with_deg:
            _zero_vmem_1d(deg_v, N_PAD)

        pltpu.sync_copy(sidx_hbm.at[s], sidx_v)
        pltpu.sync_copy(didx_hbm.at[s], didx_v)

        ones16 = jnp.ones((16,), jnp.float32)

        for k in range(5):
            pltpu.sync_copy(zbuf_v, table_sh.at[pl.ds((s * 5 + k) * 128, 128)])
        plsc.subcore_barrier()

        def grp(g, _):
            for hbm, cc in ((h0_hbm, 0), (h1_hbm, 1)):
                @pl.when(c == cc)
                def _(hbm=hbm):
                    descs = [
                        pltpu.async_copy(
                            hbm.at[sidx_v.at[g * 4 + b]], rows[b], gsem)
                        for b in range(4)]
                    for d in descs:
                        d.wait()
            sdescs = [
                pltpu.async_copy(
                    rows[b], table_sh.at[didx_v.at[g * 4 + b]], ssem,
                    add=True)
                for b in range(4)]
            for d in sdescs:
                d.wait()
            if with_deg:
                def dcount(k, _):
                    idx16 = didx_v[g * 4 + k // 8, pl.ds((k % 8) * 16, 16)]
                    plsc.addupdate_scatter(deg_v, [idx16], ones16)
                    return 0
                lax.fori_loop(0, 32, dcount, 0)
            return 0

        lax.fori_loop(0, 20, grp, 0)

        plsc.subcore_barrier()
        pltpu.sync_copy(table_sh.at[pl.ds(s * 640, 640)],
                        aggr_out.at[c].at[pl.ds(s * 640, 640)])
        if with_deg:
            pltpu.sync_copy(deg_v, deg_out.at[c].at[s])

    return pl.kernel(
        body, out_type=out_type, mesh=mesh, scratch_types=scratch,
        compiler_params=pltpu.CompilerParams(needs_layout_passes=False,
                                             use_tc_tiling_on_sc=False))


def _make_narrow_aggr():
    mesh = plsc.VectorSubcoreMesh(core_axis_name="c", subcore_axis_name="s",
                                  num_cores=2, num_subcores=16)

    scratch = [
        pltpu.VMEM((40, 128), jnp.int32),
        pltpu.VMEM((40, 128), jnp.int32),
        [pltpu.VMEM((128, CODE_PAD), jnp.bfloat16) for _ in range(4)],
        pltpu.VMEM((320, CODE_PAD), jnp.bfloat16),
        pltpu.VMEM_SHARED((N_PAD, CODE_PAD), jnp.bfloat16),
        pltpu.SemaphoreType.DMA,
        pltpu.SemaphoreType.DMA,
    ]

    def body(tbl_hbm, sidx_hbm, didx_hbm, aggr_out,
             sidx_v, didx_v, rows, zbuf_v, table_sh, gsem, ssem):
        c = lax.axis_index("c")
        s = lax.axis_index("s")
        w = c * 16 + s

        _zero_vmem_bf16(zbuf_v, 320, CODE_PAD)
        pltpu.sync_copy(zbuf_v, table_sh.at[pl.ds(s * 320, 320)])
        pltpu.sync_copy(zbuf_v, table_sh.at[pl.ds(5120 + s * 320, 320)])

        pltpu.sync_copy(sidx_hbm.at[w], sidx_v)
        pltpu.sync_copy(didx_hbm.at[w], didx_v)
        plsc.subcore_barrier()

        def grp(g, _):
            descs = [
                pltpu.async_copy(tbl_hbm.at[sidx_v.at[g * 4 + b]], rows[b],
                                 gsem)
                for b in range(4)]
            for d in descs:
                d.wait()
            sdescs = [
                pltpu.async_copy(rows[b], table_sh.at[didx_v.at[g * 4 + b]],
                                 ssem, add=True)
                for b in range(4)]
            for d in sdescs:
                d.wait()
            return 0

        lax.fori_loop(0, 10, grp, 0)

        plsc.subcore_barrier()
        pltpu.sync_copy(table_sh.at[pl.ds(s * 640, 640)],
                        aggr_out.at[c].at[pl.ds(s * 640, 640)])

    return pl.kernel(
        body,
        out_type=[jax.ShapeDtypeStruct((2, N_PAD, CODE_PAD), jnp.bfloat16)],
        mesh=mesh, scratch_types=scratch,
        compiler_params=pltpu.CompilerParams(needs_layout_passes=False,
                                             use_tc_tiling_on_sc=False))


def _t1_body(x_ref, a_ref, degs_ref, wl1_ref, wr1_ref, b1_ref,
             wl2_ref, wr2_ref, b2_ref, p2_ref, r2_ref, invd_ref):
    deg = jnp.sum(degs_ref[...], axis=0)
    invd = 1.0 / jnp.clip(deg, 1.0, None)
    invd2 = invd[:, None]
    aggr = sum(jnp.dot(a_ref[i].astype(jnp.float32), wl1_ref[i],
                       preferred_element_type=jnp.float32)
               for i in range(2))
    h1 = jax.nn.relu(aggr * invd2
                     + jnp.dot(x_ref[...], wr1_ref[...],
                               preferred_element_type=jnp.float32)
                     + b1_ref[...])
    p2_ref[...] = jnp.dot(h1, wl2_ref[...],
                          preferred_element_type=jnp.float32
                          ).astype(jnp.bfloat16)
    r2_ref[...] = (jnp.dot(h1, wr2_ref[...], preferred_element_type=jnp.float32)
                   + b2_ref[...])
    invd_ref[...] = invd2


def _t2_body(a2_ref, r2_ref, invd_ref, g_ref, z_ref, zb_ref):
    a2 = a2_ref[0].astype(jnp.float32) + a2_ref[1].astype(jnp.float32)
    t = a2 * invd_ref[...] + r2_ref[...] + g_ref[...]
    parts = []
    for grp in range(2):
        sl = t[:, grp * 10:(grp + 1) * 10]
        m = jnp.max(sl, axis=1, keepdims=True)
        e = jnp.exp(sl - m)
        parts.append(e / jnp.sum(e, axis=1, keepdims=True))
    parts.append(jnp.zeros((N_PAD, CODE_PAD - 20), jnp.float32))
    z = jnp.concatenate(parts, axis=1)
    z_ref[...] = z
    zb_ref[...] = z.astype(jnp.bfloat16)


def _t3_body(a3_ref, z_ref, invd_ref, wl3_ref, wr3_ref, b3_ref,
             h3f_ref, h3b_ref):
    a3 = a3_ref[0].astype(jnp.float32) + a3_ref[1].astype(jnp.float32)
    aggr = jnp.dot(a3 * invd_ref[...], wl3_ref[...],
                   preferred_element_type=jnp.float32)
    h3 = jax.nn.relu(aggr
                     + jnp.dot(z_ref[...], wr3_ref[...],
                               preferred_element_type=jnp.float32)
                     + b3_ref[...])
    h3f_ref[...] = h3
    for i in range(2):
        h3b_ref[i] = h3[:, i * HALF:(i + 1) * HALF].astype(jnp.bfloat16)


def _t4_body(a4_ref, h3f_ref, invd_ref, wl4_ref, wr4_ref, b4_ref, out_ref):
    aggr = sum(jnp.dot(a4_ref[i].astype(jnp.float32), wl4_ref[i],
                       preferred_element_type=jnp.float32)
               for i in range(2))
    rec = jnp.dot(h3f_ref[...], wr4_ref[...],
                  preferred_element_type=jnp.float32)
    out_ref[...] = aggr * invd_ref[...] + rec + b4_ref[...]


def _tc_call(body, out_shapes):
    return pl.pallas_call(body, out_shape=out_shapes)


ROWB = 2560


def _full(shape):
    nd = len(shape)
    return pl.BlockSpec(shape, lambda i: (0,) * nd)


def _rows(shape):
    nd = len(shape)
    if nd == 2:
        return pl.BlockSpec((ROWB, shape[1]), lambda i: (i, 0))
    return pl.BlockSpec((shape[0], ROWB, shape[2]), lambda i: (0, i, 0))


def kernel(x, edge_index, W_l1, W_r1, b1, W_l2, W_r2, b2,
           W_l3, W_r3, b3, W_l4, W_r4, b4):
    f32 = jnp.float32
    src = edge_index[0].astype(jnp.int32)
    dst = edge_index[1].astype(jnp.int32)
    src_p = jnp.concatenate(
        [src, jnp.zeros((E_PAD - E_EDGES,), jnp.int32)])
    dst_p = jnp.concatenate(
        [dst, jnp.full((E_PAD - E_EDGES,), DUMP_ROW, jnp.int32)])
    sidx16 = src_p.reshape(16, 80, 128)
    didx16 = dst_p.reshape(16, 80, 128)
    sidx32 = src_p.reshape(32, 40, 128)
    didx32 = dst_p.reshape(32, 40, 128)

    x_p = jnp.pad(x.astype(f32), ((0, N_PAD - N_NODES), (0, 0)))
    x_b = x_p.astype(jnp.bfloat16)
    xh = [x_b[:, :HALF], x_b[:, HALF:]]

    wl1s = W_l1.reshape(2, HALF, HID)
    wl2p = jnp.pad(W_l2, ((0, 0), (0, CODE_PAD - 20)))
    wr2p = jnp.pad(W_r2, ((0, 0), (0, CODE_PAD - 20)))
    b2p = jnp.pad(b2, (0, CODE_PAD - 20))[None, :]
    wl3p = jnp.pad(W_l3, ((0, CODE_PAD - 20), (0, 0)))
    wr3p = jnp.pad(W_r3, ((0, CODE_PAD - 20), (0, 0)))
    wl4s = W_l4.reshape(2, HALF, IN_DIM)

    u = jax.random.uniform(jax.random.key(42), (N_NODES, 2, 10), dtype=f32)
    g = -jnp.log(-jnp.log(u + 1e-20)).reshape(N_NODES, 20)
    g_p = jnp.pad(g, ((0, N_PAD - N_NODES), (0, CODE_PAD - 20)))

    wide_deg = _make_wide_aggr(with_deg=True)
    narrow = _make_narrow_aggr()
    wide = _make_wide_aggr(with_deg=False)

    aggr1, degs = wide_deg(xh[0], xh[1], sidx16, didx16)

    p2, r2, invd = pl.pallas_call(
        _t1_body,
        grid=(N_PAD // ROWB,),
        in_specs=[_rows((N_PAD, HID)), _rows((2, N_PAD, HALF)),
                  pl.BlockSpec((16, ROWB), lambda i: (0, i)),
                  _full((2, HALF, HID)), _full((HID, HID)), _full((1, HID)),
                  _full((HID, CODE_PAD)), _full((HID, CODE_PAD)),
                  _full((1, CODE_PAD))],
        out_specs=[_rows((N_PAD, CODE_PAD)), _rows((N_PAD, CODE_PAD)),
                   _rows((N_PAD, 1))],
        out_shape=[jax.ShapeDtypeStruct((N_PAD, CODE_PAD), jnp.bfloat16),
                   jax.ShapeDtypeStruct((N_PAD, CODE_PAD), f32),
                   jax.ShapeDtypeStruct((N_PAD, 1), f32)],
    )(x_p, aggr1, degs[0], wl1s, W_r1, b1[None, :], wl2p, wr2p, b2p)

    (a2,) = narrow(p2, sidx32, didx32)

    z, z_b = _tc_call(
        _t2_body, [jax.ShapeDtypeStruct((N_PAD, CODE_PAD), f32),
                   jax.ShapeDtypeStruct((N_PAD, CODE_PAD), jnp.bfloat16)]
    )(a2, r2, invd, g_p)

    (a3,) = narrow(z_b, sidx32, didx32)

    h3f, h3b = _tc_call(
        _t3_body, [jax.ShapeDtypeStruct((N_PAD, HID), f32),
                   jax.ShapeDtypeStruct((2, N_PAD, HALF), jnp.bfloat16)]
    )(a3, z, invd, wl3p, wr3p, b3[None, :])

    (aggr4,) = wide(h3b[0], h3b[1], sidx16, didx16)

    (out,) = pl.pallas_call(
        _t4_body,
        grid=(N_PAD // ROWB,),
        in_specs=[_rows((2, N_PAD, HALF)), _rows((N_PAD, HID)),
                  _rows((N_PAD, 1)), _full((2, HALF, IN_DIM)),
                  _full((HID, IN_DIM)), _full((1, IN_DIM))],
        out_specs=[_rows((N_PAD, IN_DIM))],
        out_shape=[jax.ShapeDtypeStruct((N_PAD, IN_DIM), f32)],
    )(aggr4, h3f, invd, wl4s, W_r4, b4[None, :])

    return out[:N_NODES]
